# Initial kernel scaffold; baseline (speedup 1.0000x reference)
import functools

import jax
import jax.numpy as jnp
from jax import lax
from jax.experimental import pallas as pl
from jax.experimental.pallas import tpu as pltpu
from jax.experimental.pallas import tpu_sc as plsc

_B, _N, _DC, _DP = 16, 100000, 3, 8
_K = 16
_L = 16
_CHUNK = 10000
_NCHUNK = _N // _CHUNK
_G = 5
_NGROUP = _CHUNK // (_L * _G)
_INF = jnp.float32(jnp.inf)


def _merge16(bd, bi, d, i):
  sd, si = plsc.sort_key_val(d, i)
  rd = lax.rev(sd, (0,))
  ri = lax.rev(si, (0,))
  take_old = bd <= rd
  nd = jnp.where(take_old, bd, rd)
  ni = jnp.where(take_old, bi, ri)
  return plsc.sort_key_val(nd, ni)


def _sc_topk_features(latent_grid_point, coords, params):
  mesh = plsc.VectorSubcoreMesh(core_axis_name="c", subcore_axis_name="s")

  @functools.partial(
      pl.kernel,
      out_type=jax.ShapeDtypeStruct((_B, 16), jnp.float32),
      mesh=mesh,
      scratch_types=[
          pltpu.VMEM((_CHUNK, _DC), jnp.float32),
          pltpu.VMEM((_CHUNK, _DC), jnp.float32),
          pltpu.VMEM((2,), jnp.float32),
          pltpu.VMEM((_K,), jnp.int32),
          pltpu.VMEM((_K, _DC), jnp.float32),
          pltpu.VMEM((_K, _DP), jnp.float32),
          pltpu.VMEM((16,), jnp.float32),
          pltpu.SemaphoreType.DMA,
          pltpu.SemaphoreType.DMA,
          pltpu.SemaphoreType.DMA,
      ],
  )
  def scan_kernel(lgp_hbm, coords_hbm, params_hbm, out_hbm,
                  buf0, buf1, lgp_v, idx_v, crows, prows, feat_v,
                  sem0, sem1, gsem):
    c = lax.axis_index("c")
    s = lax.axis_index("s")
    wid = s * 2 + c

    @pl.when(wid < _B)
    def _():
      b = wid
      pltpu.sync_copy(lgp_hbm, lgp_v)
      gx = lgp_v[0]
      gy = lgp_v[1]

      iota = lax.iota(jnp.int32, _L)
      bufs = (buf0, buf1)
      sems = (sem0, sem1)

      cp = pltpu.async_copy(
          coords_hbm.at[b, pl.ds(0, _CHUNK), :], buf0, sem0)

      best_d = jnp.full((_L,), _INF, jnp.float32)
      best_i = jnp.zeros((_L,), jnp.int32)
      thr = _INF

      def dists(buf, g, u):
        r = (g * _G + u) * _L + iota
        x = plsc.load_gather(buf, [r, jnp.zeros((_L,), jnp.int32)])
        y = plsc.load_gather(buf, [r, jnp.ones((_L,), jnp.int32)])
        dx = x - gx
        dy = y - gy
        return dx * dx + dy * dy

      for ck in range(_NCHUNK):
        buf = bufs[ck % 2]
        cp.wait()
        if ck + 1 < _NCHUNK:
          cp = pltpu.async_copy(
              coords_hbm.at[b, pl.ds((ck + 1) * _CHUNK, _CHUNK), :],
              bufs[(ck + 1) % 2], sems[(ck + 1) % 2])

        cbase = ck * _CHUNK

        def group_body(g, carry, buf=buf, cbase=cbase):
          bd, bi, th = carry
          ds = [dists(buf, g, u) for u in range(_G)]
          gmin = ds[0]
          for u in range(1, _G):
            gmin = jnp.minimum(gmin, ds[u])
          hit = jnp.min(gmin) < th

          def slow(bd, bi, th, g, *ds):
            for u in range(_G):
              du = ds[u]
              iu = cbase + (g * _G + u) * _L + iota

              def do_merge(bd, bi, du=du, iu=iu):
                nd, ni = _merge16(bd, bi, du, iu)
                return nd, ni, jnp.max(nd)

              bd, bi, th = lax.cond(
                  jnp.min(du) < th,
                  do_merge,
                  lambda bd, bi, th=th: (bd, bi, th),
                  bd, bi)
            return bd, bi, th

          return lax.cond(
              hit, slow, lambda bd, bi, th, g, *ds: (bd, bi, th),
              bd, bi, th, g, *ds)

        best_d, best_i, thr = lax.fori_loop(
            0, _NGROUP, group_body, (best_d, best_i, thr))

      idx_v[...] = best_i
      pltpu.async_copy(coords_hbm.at[b].at[idx_v], crows, gsem).wait()
      pltpu.async_copy(params_hbm.at[b].at[idx_v], prows, gsem).wait()

      inv_k = jnp.float32(1.0 / _K)
      col = lambda ref, j: plsc.load_gather(
          ref, [iota, jnp.full((_L,), j, jnp.int32)])
      feat_v[...] = jnp.zeros((16,), jnp.float32)
      feat_v[0] = gx
      feat_v[1] = gy
      feat_v[2] = jnp.sum(col(crows, 0)) * inv_k
      feat_v[3] = jnp.sum(col(crows, 1)) * inv_k
      for j in range(_DP):
        feat_v[4 + j] = jnp.sum(col(prows, j)) * inv_k
      pltpu.sync_copy(feat_v, out_hbm.at[b])

  return scan_kernel(latent_grid_point, coords, params)


def _mlp_kernel(feat_ref, w1_ref, b1_ref, w2_ref, b2_ref, w3_ref, b3_ref,
                out_ref):
  x = feat_ref[:, :12]
  h = jnp.maximum(
      jnp.dot(x, w1_ref[...], preferred_element_type=jnp.float32)
      + b1_ref[...], 0.0)
  h = jnp.maximum(
      jnp.dot(h, w2_ref[...], preferred_element_type=jnp.float32)
      + b2_ref[...], 0.0)
  out_ref[...] = (
      jnp.dot(h, w3_ref[...], preferred_element_type=jnp.float32)
      + b3_ref[...])


def kernel(latent_grid_point, coords, params, W1, b1, W2, b2, W3, b3):
  feat = _sc_topk_features(latent_grid_point, coords, params)
  out = pl.pallas_call(
      _mlp_kernel,
      out_shape=jax.ShapeDtypeStruct((_B, 64), jnp.float32),
  )(feat, W1, b1.reshape(1, 128), W2, b2.reshape(1, 128),
    W3, b3.reshape(1, 64))
  return out


# speedup vs baseline: 1.1031x; 1.1031x over previous
import functools

import jax
import jax.numpy as jnp
from jax import lax
from jax.experimental import pallas as pl
from jax.experimental.pallas import tpu as pltpu
from jax.experimental.pallas import tpu_sc as plsc

_B, _N, _DC, _DP = 16, 100000, 3, 8
_K = 16
_L = 16
_CHUNK = 10000
_C3 = _CHUNK * _DC
_NCHUNK = _N // _CHUNK
_G = 5
_NGROUP = _CHUNK // (_L * _G)
_INF = float("inf")


def _merge16(bd, bi, d, i):
  sd, si = plsc.sort_key_val(d, i)
  rd = lax.rev(sd, (0,))
  ri = lax.rev(si, (0,))
  take_old = bd <= rd
  nd = jnp.where(take_old, bd, rd)
  ni = jnp.where(take_old, bi, ri)
  return plsc.sort_key_val(nd, ni)


def _sc_topk_features(latent_grid_point, coords_flat, params_flat):
  mesh = plsc.VectorSubcoreMesh(
      core_axis_name="c", subcore_axis_name="s", num_cores=2, num_subcores=16)

  @functools.partial(
      pl.kernel,
      out_type=jax.ShapeDtypeStruct((_B, 16), jnp.float32),
      mesh=mesh,
      compiler_params=pltpu.CompilerParams(
          use_tc_tiling_on_sc=False, needs_layout_passes=False),
      scratch_types=[
          pltpu.VMEM((_C3,), jnp.float32),
          pltpu.VMEM((_C3,), jnp.float32),
          pltpu.VMEM((16,), jnp.float32),
          pltpu.VMEM((2 * _K,), jnp.int32),
          pltpu.VMEM((_DP * _K,), jnp.int32),
          pltpu.VMEM((2 * _K,), jnp.float32),
          pltpu.VMEM((_DP * _K,), jnp.float32),
          pltpu.VMEM((16,), jnp.float32),
          pltpu.SemaphoreType.DMA,
          pltpu.SemaphoreType.DMA,
          pltpu.SemaphoreType.DMA,
      ],
  )
  def scan_kernel(lgp_hbm, coords_hbm, params_hbm, out_hbm,
                  buf0, buf1, lgp_v, idxc, idxp, cvals, pvals, feat_v,
                  sem0, sem1, gsem):
    c = lax.axis_index("c")
    s = lax.axis_index("s")
    wid = s * 2 + c

    @pl.when(wid < _B)
    def _():
      b = wid
      pltpu.sync_copy(lgp_hbm, lgp_v.at[pl.ds(0, 2)])
      lv = lgp_v[...]
      gx = lv[0]
      gy = lv[1]

      iota = lax.iota(jnp.int32, _L)
      iota3 = iota * 3
      bufs = (buf0, buf1)
      sems = (sem0, sem1)

      cp = pltpu.async_copy(coords_hbm.at[b, pl.ds(0, _C3)], buf0, sem0)

      best_d = jnp.full((_L,), _INF, jnp.float32)
      best_i = jnp.zeros((_L,), jnp.int32)
      thr = _INF

      def dists(buf, g, u):
        base3 = (g * _G + u) * (_L * 3)
        x = plsc.load_gather(buf, [base3 + iota3])
        y = plsc.load_gather(buf, [base3 + iota3 + 1])
        dx = x - gx
        dy = y - gy
        return dx * dx + dy * dy

      for ck in range(_NCHUNK):
        buf = bufs[ck % 2]
        cp.wait()
        if ck + 1 < _NCHUNK:
          cp = pltpu.async_copy(
              coords_hbm.at[b, pl.ds((ck + 1) * _C3, _C3)],
              bufs[(ck + 1) % 2], sems[(ck + 1) % 2])

        cbase = ck * _CHUNK

        def group_body(g, carry, buf=buf, cbase=cbase):
          bd, bi, th = carry
          ds = [dists(buf, g, u) for u in range(_G)]
          gmin = ds[0]
          for u in range(1, _G):
            gmin = jnp.minimum(gmin, ds[u])
          hit = jnp.min(gmin) < th

          def slow(bd, bi, th, g, *ds):
            for u in range(_G):
              du = ds[u]
              iu = cbase + (g * _G + u) * _L + iota

              def do_merge(bd, bi, du=du, iu=iu):
                nd, ni = _merge16(bd, bi, du, iu)
                return nd, ni, jnp.max(nd)

              bd, bi, th = lax.cond(
                  jnp.min(du) < th,
                  do_merge,
                  lambda bd, bi, th=th: (bd, bi, th),
                  bd, bi)
            return bd, bi, th

          return lax.cond(
              hit, slow, lambda bd, bi, th, g, *ds: (bd, bi, th),
              bd, bi, th, g, *ds)

        best_d, best_i, thr = lax.fori_loop(
            0, _NGROUP, group_body, (best_d, best_i, thr))

      i3 = best_i * 3
      idxc[pl.ds(0, _L)] = i3
      idxc[pl.ds(_L, _L)] = i3 + 1
      i8 = best_i * _DP
      for j in range(_DP):
        idxp[pl.ds(j * _L, _L)] = i8 + j
      gc = pltpu.async_copy(coords_hbm.at[b].at[idxc], cvals, gsem)
      gp = pltpu.async_copy(params_hbm.at[b].at[idxp], pvals, gsem)
      gc.wait()
      gp.wait()

      inv_k = jnp.float32(1.0 / _K)
      vals = [gx, gy,
              jnp.sum(cvals[pl.ds(0, _L)]) * inv_k,
              jnp.sum(cvals[pl.ds(_L, _L)]) * inv_k]
      vals += [jnp.sum(pvals[pl.ds(j * _L, _L)]) * inv_k
               for j in range(_DP)]
      feat = jnp.zeros((16,), jnp.float32)
      for k, v in enumerate(vals):
        feat = jnp.where(iota == k, v, feat)
      feat_v[...] = feat
      pltpu.sync_copy(feat_v, out_hbm.at[b])

  return scan_kernel(latent_grid_point, coords_flat, params_flat)


def _mlp_kernel(feat_ref, w1_ref, b1_ref, w2_ref, b2_ref, w3_ref, b3_ref,
                out_ref):
  x = feat_ref[:, :12]
  h = jnp.maximum(
      jnp.dot(x, w1_ref[...], preferred_element_type=jnp.float32)
      + b1_ref[...], 0.0)
  h = jnp.maximum(
      jnp.dot(h, w2_ref[...], preferred_element_type=jnp.float32)
      + b2_ref[...], 0.0)
  out_ref[...] = (
      jnp.dot(h, w3_ref[...], preferred_element_type=jnp.float32)
      + b3_ref[...])


def kernel(latent_grid_point, coords, params, W1, b1, W2, b2, W3, b3):
  coords_flat = coords.reshape(_B, _N * _DC)
  params_flat = params.reshape(_B, _N * _DP)
  feat = _sc_topk_features(latent_grid_point, coords_flat, params_flat)
  out = pl.pallas_call(
      _mlp_kernel,
      out_shape=jax.ShapeDtypeStruct((_B, 64), jnp.float32),
  )(feat, W1, b1.reshape(1, 128), W2, b2.reshape(1, 128),
    W3, b3.reshape(1, 64))
  return out


# speedup vs baseline: 11.7591x; 10.6602x over previous
import functools

import jax
import jax.numpy as jnp
from jax import lax
from jax.experimental import pallas as pl
from jax.experimental.pallas import tpu as pltpu
from jax.experimental.pallas import tpu_sc as plsc

_B, _N, _DC, _DP = 16, 100000, 3, 8
_K = 16
_L = 16
_NPAD = 102400
_ROWS = 8
_RN = _NPAD // _ROWS
_CN = 3200
_NCK = _RN // _CN
_G = 5
_GPTS = _G * _L
_NGROUP = _ROWS * _CN // _GPTS
_INF = float("inf")
_PAD_VAL = 1e30


def _merge16(bd, bi, d, i):
  sd, si = plsc.sort_key_val(d, i)
  rd = lax.rev(sd, (0,))
  ri = lax.rev(si, (0,))
  take_old = bd <= rd
  nd = jnp.where(take_old, bd, rd)
  ni = jnp.where(take_old, bi, ri)
  return plsc.sort_key_val(nd, ni)


def _sc_topk_idx(xs, ys):
  mesh = plsc.VectorSubcoreMesh(
      core_axis_name="c", subcore_axis_name="s", num_cores=2, num_subcores=16)

  @functools.partial(
      pl.kernel,
      out_type=jax.ShapeDtypeStruct((_B * _K,), jnp.int32),
      mesh=mesh,
      compiler_params=pltpu.CompilerParams(
          use_tc_tiling_on_sc=True, needs_layout_passes=False),
      scratch_types=[
          pltpu.VMEM((_ROWS, _CN), jnp.float32),
          pltpu.VMEM((_ROWS, _CN), jnp.float32),
          pltpu.VMEM((_ROWS, _CN), jnp.float32),
          pltpu.VMEM((_ROWS, _CN), jnp.float32),
          pltpu.VMEM((_K,), jnp.int32),
          pltpu.SemaphoreType.DMA,
          pltpu.SemaphoreType.DMA,
          pltpu.SemaphoreType.DMA,
          pltpu.SemaphoreType.DMA,
      ],
  )
  def scan_kernel(xs_hbm, ys_hbm, out_hbm,
                  xbuf0, xbuf1, ybuf0, ybuf1, idx_v,
                  xsem0, xsem1, ysem0, ysem1):
    c = lax.axis_index("c")
    s = lax.axis_index("s")
    wid = s * 2 + c

    @pl.when(wid < _B)
    def _():
      b = wid
      iota = lax.iota(jnp.int32, _L)
      xbufs = (xbuf0, xbuf1)
      ybufs = (ybuf0, ybuf1)
      xsems = (xsem0, xsem1)
      ysems = (ysem0, ysem1)

      cpx = pltpu.async_copy(xs_hbm.at[b, :, pl.ds(0, _CN)], xbuf0, xsem0)
      cpy = pltpu.async_copy(ys_hbm.at[b, :, pl.ds(0, _CN)], ybuf0, ysem0)

      best_d = jnp.full((_L,), _INF, jnp.float32)
      best_i = jnp.zeros((_L,), jnp.int32)
      thr = _INF

      for ck in range(_NCK):
        xb = xbufs[ck % 2]
        yb = ybufs[ck % 2]
        cpx.wait()
        cpy.wait()
        if ck + 1 < _NCK:
          cpx = pltpu.async_copy(
              xs_hbm.at[b, :, pl.ds((ck + 1) * _CN, _CN)],
              xbufs[(ck + 1) % 2], xsems[(ck + 1) % 2])
          cpy = pltpu.async_copy(
              ys_hbm.at[b, :, pl.ds((ck + 1) * _CN, _CN)],
              ybufs[(ck + 1) % 2], ysems[(ck + 1) % 2])

        cbase = ck * _CN

        def group_body(g, carry, xb=xb, yb=yb, cbase=cbase):
          del g
          bd, bi, th, r, cc = carry

          def dists(u):
            x = xb[r, pl.ds(cc + u * _L, _L)]
            y = yb[r, pl.ds(cc + u * _L, _L)]
            return x * x + y * y

          ds = [dists(u) for u in range(_G)]
          gmin = ds[0]
          for u in range(1, _G):
            gmin = jnp.minimum(gmin, ds[u])
          hit = jnp.min(gmin) < th
          pbase = r * _RN + cbase + cc

          def slow(bd, bi, th, pbase, *ds):
            for u in range(_G):
              du = ds[u]
              iu = pbase + u * _L + iota

              def do_merge(bd, bi, du=du, iu=iu):
                nd, ni = _merge16(bd, bi, du, iu)
                return nd, ni, jnp.max(nd)

              bd, bi, th = lax.cond(
                  jnp.min(du) < th,
                  do_merge,
                  lambda bd, bi, th=th: (bd, bi, th),
                  bd, bi)
            return bd, bi, th

          bd, bi, th = lax.cond(
              hit, slow, lambda bd, bi, th, pbase, *ds: (bd, bi, th),
              bd, bi, th, pbase, *ds)

          cc2 = cc + _GPTS
          wrap = cc2 >= _CN
          r2 = jnp.where(wrap, r + 1, r)
          cc3 = jnp.where(wrap, 0, cc2)
          return bd, bi, th, r2, cc3

        best_d, best_i, thr, _, _ = lax.fori_loop(
            0, _NGROUP, group_body,
            (best_d, best_i, thr, jnp.int32(0), jnp.int32(0)))

      idx_v[...] = best_i
      pltpu.sync_copy(idx_v, out_hbm.at[pl.ds(b * _K, _K)])

  return scan_kernel(xs, ys)


def _mlp_kernel(lgp_ref, nc_ref, np_ref, w1_ref, b1_ref, w2_ref, b2_ref,
                w3_ref, b3_ref, out_ref):
  inv_k = jnp.float32(1.0 / _K)
  mean_xy = jnp.sum(nc_ref[...], axis=1) * inv_k
  mean_p = jnp.sum(np_ref[...], axis=1) * inv_k
  lgp = jnp.broadcast_to(lgp_ref[...], (_B, 2))
  x = jnp.concatenate([lgp, mean_xy, mean_p], axis=1)
  h = jnp.maximum(
      jnp.dot(x, w1_ref[...], preferred_element_type=jnp.float32)
      + b1_ref[...], 0.0)
  h = jnp.maximum(
      jnp.dot(h, w2_ref[...], preferred_element_type=jnp.float32)
      + b2_ref[...], 0.0)
  out_ref[...] = (
      jnp.dot(h, w3_ref[...], preferred_element_type=jnp.float32)
      + b3_ref[...])


def kernel(latent_grid_point, coords, params, W1, b1, W2, b2, W3, b3):
  gx = latent_grid_point[0]
  gy = latent_grid_point[1]
  dx = gx - coords[:, :, 0]
  dy = gy - coords[:, :, 1]
  pad = ((0, 0), (0, _NPAD - _N))
  xs = jnp.pad(dx, pad, constant_values=_PAD_VAL).reshape(_B, _ROWS, _RN)
  ys = jnp.pad(dy, pad, constant_values=_PAD_VAL).reshape(_B, _ROWS, _RN)
  idx = _sc_topk_idx(xs, ys).reshape(_B, _K)

  idxe = idx[:, :, None]
  ncoords = jnp.take_along_axis(coords[:, :, :2], idxe, axis=1)
  nparams = jnp.take_along_axis(params, idxe, axis=1)

  out = pl.pallas_call(
      _mlp_kernel,
      out_shape=jax.ShapeDtypeStruct((_B, 64), jnp.float32),
  )(latent_grid_point.reshape(1, 2), ncoords, nparams,
    W1, b1.reshape(1, 128), W2, b2.reshape(1, 128), W3, b3.reshape(1, 64))
  return out


# speedup vs baseline: 14.1572x; 1.2039x over previous
import functools

import jax
import jax.numpy as jnp
from jax import lax
from jax.experimental import pallas as pl
from jax.experimental.pallas import tpu as pltpu
from jax.experimental.pallas import tpu_sc as plsc

_B, _N, _DC, _DP = 16, 100000, 3, 8
_K = 16
_L = 16
_NPAD = 102400
_ROWS = 8
_RN = _NPAD // _ROWS
_HN = _RN // 2
_CN = 3200
_NCK = _HN // _CN
_G = 10
_GPTS = _G * _L
_NGROUP = _ROWS * _CN // _GPTS
_INF = float("inf")
_PAD_VAL = 1e30


def _merge16(bd, bi, d, i):
  sd, si = plsc.sort_key_val(d, i)
  rd = lax.rev(sd, (0,))
  ri = lax.rev(si, (0,))
  take_old = bd <= rd
  nd = jnp.where(take_old, bd, rd)
  ni = jnp.where(take_old, bi, ri)
  return plsc.sort_key_val(nd, ni)


def _sc_topk_idx(lgp, xs, ys):
  mesh = plsc.VectorSubcoreMesh(
      core_axis_name="c", subcore_axis_name="s", num_cores=2, num_subcores=16)

  @functools.partial(
      pl.kernel,
      out_type=jax.ShapeDtypeStruct((_B * _K,), jnp.int32),
      mesh=mesh,
      compiler_params=pltpu.CompilerParams(
          use_tc_tiling_on_sc=True, needs_layout_passes=False),
      scratch_types=[
          pltpu.VMEM((_ROWS, _CN), jnp.float32),
          pltpu.VMEM((_ROWS, _CN), jnp.float32),
          pltpu.VMEM((_ROWS, _CN), jnp.float32),
          pltpu.VMEM((_ROWS, _CN), jnp.float32),
          pltpu.VMEM((16,), jnp.float32),
          pltpu.VMEM((_K,), jnp.float32),
          pltpu.VMEM((_K,), jnp.int32),
          pltpu.VMEM((_K,), jnp.float32),
          pltpu.VMEM((_K,), jnp.int32),
          pltpu.VMEM_SHARED((16, _K), jnp.float32),
          pltpu.VMEM_SHARED((16, _K), jnp.int32),
          pltpu.SemaphoreType.DMA,
          pltpu.SemaphoreType.DMA,
          pltpu.SemaphoreType.DMA,
          pltpu.SemaphoreType.DMA,
          pltpu.SemaphoreType.DMA,
      ],
  )
  def scan_kernel(lgp_hbm, xs_hbm, ys_hbm, out_hbm,
                  xbuf0, xbuf1, ybuf0, ybuf1, lgp_v,
                  dv, iv, dv2, iv2, dsh, ish,
                  xsem0, xsem1, ysem0, ysem1, gsem):
    c = lax.axis_index("c")
    s = lax.axis_index("s")
    b = c * 8 + lax.rem(s, 8)
    h = s // 8

    iota = lax.iota(jnp.int32, _L)
    xbufs = (xbuf0, xbuf1)
    ybufs = (ybuf0, ybuf1)
    xsems = (xsem0, xsem1)
    ysems = (ysem0, ysem1)

    pltpu.sync_copy(lgp_hbm, lgp_v.at[pl.ds(0, 2)])
    lv = lgp_v[...]
    gx = lv[0]
    gy = lv[1]

    hbase = h * _HN
    cpx = pltpu.async_copy(
        xs_hbm.at[b, :, pl.ds(hbase, _CN)], xbuf0, xsem0)
    cpy = pltpu.async_copy(
        ys_hbm.at[b, :, pl.ds(hbase, _CN)], ybuf0, ysem0)

    best_d = jnp.full((_L,), _INF, jnp.float32)
    best_i = jnp.zeros((_L,), jnp.int32)
    thr = _INF

    for ck in range(_NCK):
      xb = xbufs[ck % 2]
      yb = ybufs[ck % 2]
      cpx.wait()
      cpy.wait()
      if ck + 1 < _NCK:
        cpx = pltpu.async_copy(
            xs_hbm.at[b, :, pl.ds(hbase + (ck + 1) * _CN, _CN)],
            xbufs[(ck + 1) % 2], xsems[(ck + 1) % 2])
        cpy = pltpu.async_copy(
            ys_hbm.at[b, :, pl.ds(hbase + (ck + 1) * _CN, _CN)],
            ybufs[(ck + 1) % 2], ysems[(ck + 1) % 2])

      cbase = hbase + ck * _CN

      def group_body(g, carry, xb=xb, yb=yb, cbase=cbase):
        del g
        bd, bi, th, r, cc = carry

        def dists(u):
          x = xb[r, pl.ds(cc + u * _L, _L)]
          y = yb[r, pl.ds(cc + u * _L, _L)]
          dx = x - gx
          dy = y - gy
          return dx * dx + dy * dy

        ds = [dists(u) for u in range(_G)]
        gmin = ds[0]
        for u in range(1, _G):
          gmin = jnp.minimum(gmin, ds[u])
        hit = jnp.min(gmin) < th
        pbase = r * _RN + cbase + cc

        def slow(bd, bi, th, pbase, *ds):
          for u in range(_G):
            du = ds[u]
            iu = pbase + u * _L + iota

            def do_merge(bd, bi, du=du, iu=iu):
              nd, ni = _merge16(bd, bi, du, iu)
              return nd, ni, jnp.max(nd)

            bd, bi, th = lax.cond(
                jnp.min(du) < th,
                do_merge,
                lambda bd, bi, th=th: (bd, bi, th),
                bd, bi)
          return bd, bi, th

        bd, bi, th = lax.cond(
            hit, slow, lambda bd, bi, th, pbase, *ds: (bd, bi, th),
            bd, bi, th, pbase, *ds)

        cc2 = cc + _GPTS
        wrap = cc2 >= _CN
        r2 = jnp.where(wrap, r + 1, r)
        cc3 = jnp.where(wrap, 0, cc2)
        return bd, bi, th, r2, cc3

      best_d, best_i, thr, _, _ = lax.fori_loop(
          0, _NGROUP, group_body,
          (best_d, best_i, thr, jnp.int32(0), jnp.int32(0)))

    dv[...] = best_d
    iv[...] = best_i
    pltpu.sync_copy(dv, dsh.at[s])
    pltpu.sync_copy(iv, ish.at[s])
    plsc.subcore_barrier()

    @pl.when(h == 0)
    def _():
      pltpu.sync_copy(dsh.at[s + 8], dv2)
      pltpu.sync_copy(ish.at[s + 8], iv2)
      nd, ni = _merge16(best_d, best_i, dv2[...], iv2[...])
      iv[...] = ni
      pltpu.sync_copy(iv, out_hbm.at[pl.ds(b * _K, _K)])

  return scan_kernel(lgp, xs, ys)


def _mlp_kernel(lgp_ref, nc_ref, np_ref, w1_ref, b1_ref, w2_ref, b2_ref,
                w3_ref, b3_ref, out_ref):
  inv_k = jnp.float32(1.0 / _K)
  mean_xy = jnp.sum(nc_ref[...], axis=1) * inv_k
  mean_p = jnp.sum(np_ref[...], axis=1) * inv_k
  lgp = jnp.broadcast_to(lgp_ref[...], (_B, 2))
  x = jnp.concatenate([lgp, mean_xy, mean_p], axis=1)
  h = jnp.maximum(
      jnp.dot(x, w1_ref[...], preferred_element_type=jnp.float32)
      + b1_ref[...], 0.0)
  h = jnp.maximum(
      jnp.dot(h, w2_ref[...], preferred_element_type=jnp.float32)
      + b2_ref[...], 0.0)
  out_ref[...] = (
      jnp.dot(h, w3_ref[...], preferred_element_type=jnp.float32)
      + b3_ref[...])


def kernel(latent_grid_point, coords, params, W1, b1, W2, b2, W3, b3):
  pad = ((0, 0), (0, _NPAD - _N))
  xs = jnp.pad(coords[:, :, 0], pad,
               constant_values=_PAD_VAL).reshape(_B, _ROWS, _RN)
  ys = jnp.pad(coords[:, :, 1], pad,
               constant_values=_PAD_VAL).reshape(_B, _ROWS, _RN)
  idx = _sc_topk_idx(latent_grid_point, xs, ys).reshape(_B, _K)

  idxe = idx[:, :, None]
  ncoords = jnp.take_along_axis(coords[:, :, :2], idxe, axis=1)
  nparams = jnp.take_along_axis(params, idxe, axis=1)

  out = pl.pallas_call(
      _mlp_kernel,
      out_shape=jax.ShapeDtypeStruct((_B, 64), jnp.float32),
  )(latent_grid_point.reshape(1, 2), ncoords, nparams,
    W1, b1.reshape(1, 128), W2, b2.reshape(1, 128), W3, b3.reshape(1, 64))
  return out
